# baseline (speedup 1.0000x reference)
import functools

import jax
import jax.numpy as jnp
from jax import lax
from jax.experimental import pallas as pl
from jax.experimental.pallas import tpu as pltpu
from jax.experimental.pallas import tpu_sc as plsc

BATCH = 16384
HIDDEN = 64
NC = 2
NS = 16
NW = NC * NS
B_PER_W = BATCH // NW
CHUNK = 128
NCHUNK = B_PER_W // CHUNK


def _sc_gather(idx_hbm, table_hbm, out_hbm, idx_v, rows_v, sem):
    wid = lax.axis_index("s") * NC + lax.axis_index("c")
    base = wid * B_PER_W
    pltpu.sync_copy(idx_hbm.at[wid], idx_v)
    copies = []
    for j in range(NCHUNK):
        copies.append(
            pltpu.async_copy(
                table_hbm.at[idx_v.at[j]],
                rows_v.at[pl.ds(j * CHUNK, CHUNK)],
                sem,
            )
        )
    for c in copies:
        c.wait()
    pltpu.sync_copy(rows_v, out_hbm.at[pl.ds(base, B_PER_W)])


@jax.jit
def _gather_rows(class_labels, table):
    idx = class_labels.astype(jnp.int32).reshape(NW, NCHUNK, CHUNK)
    mesh = plsc.VectorSubcoreMesh(core_axis_name="c", subcore_axis_name="s")
    return pl.kernel(
        _sc_gather,
        out_type=jax.ShapeDtypeStruct((BATCH, HIDDEN), jnp.float32),
        mesh=mesh,
        scratch_types=[
            pltpu.VMEM((NCHUNK, CHUNK), jnp.int32),
            pltpu.VMEM((B_PER_W, HIDDEN), jnp.float32),
            pltpu.SemaphoreType.DMA,
        ],
        compiler_params=pltpu.CompilerParams(use_tc_tiling_on_sc=False),
    )(idx, table)


def _mlp_body(emb_ref, w1_ref, b1_ref, w2_ref, b2_ref, gamma_ref, beta_ref,
              out_ref):
    x = emb_ref[...]
    h = jnp.dot(x, w1_ref[...], preferred_element_type=jnp.float32)
    h = h + b1_ref[...]
    h = 0.5 * h * (1.0 + lax.erf(h * 0.7071067811865476))
    y = jnp.dot(h, w2_ref[...], preferred_element_type=jnp.float32)
    y = y + b2_ref[...]
    mean = jnp.mean(y, axis=-1, keepdims=True)
    c = y - mean
    var = jnp.mean(c * c, axis=-1, keepdims=True)
    out_ref[...] = c * lax.rsqrt(var + 1e-5) * gamma_ref[...] + beta_ref[...]


@jax.jit
def _mlp(emb, W1, b1, W2, b2, gamma, beta):
    blk = 2048
    grid = (BATCH // blk,)
    rep2 = lambda i: (0, 0)
    return pl.pallas_call(
        _mlp_body,
        grid=grid,
        in_specs=[
            pl.BlockSpec((blk, HIDDEN), lambda i: (i, 0)),
            pl.BlockSpec((HIDDEN, 2 * HIDDEN), rep2),
            pl.BlockSpec((1, 2 * HIDDEN), rep2),
            pl.BlockSpec((2 * HIDDEN, HIDDEN), rep2),
            pl.BlockSpec((1, HIDDEN), rep2),
            pl.BlockSpec((1, HIDDEN), rep2),
            pl.BlockSpec((1, HIDDEN), rep2),
        ],
        out_specs=pl.BlockSpec((blk, HIDDEN), lambda i: (i, 0)),
        out_shape=jax.ShapeDtypeStruct((BATCH, HIDDEN), jnp.float32),
    )(emb, W1, b1.reshape(1, -1), W2, b2.reshape(1, -1),
      gamma.reshape(1, -1), beta.reshape(1, -1))


def kernel(class_labels, table, W1, b1, W2, b2, gamma, beta):
    emb = _gather_rows(class_labels, table)
    return _mlp(emb, W1, b1, W2, b2, gamma, beta)
